# baseline (speedup 1.0000x reference)
import functools

import jax
import jax.numpy as jnp
from jax import lax
from jax.experimental import pallas as pl
from jax.experimental.pallas import tpu as pltpu
from jax.experimental.pallas import tpu_sc as plsc

NUM_EMB = 100000
DIM = 4
PDIM = 2 * DIM
NPAIR = NUM_EMB // 2
W = 2048
NW = 32
L = 16


def _emb_kernel(n: int):
    per_w = n // NW
    n_wnd = per_w // W
    assert n_wnd % 2 == 0
    stage_rows = (NPAIR // 16) // 8 * 8
    tail_rows = NPAIR - 16 * stage_rows

    mesh = plsc.VectorSubcoreMesh(core_axis_name="c", subcore_axis_name="s")

    @functools.partial(
        pl.kernel,
        mesh=mesh,
        out_type=jax.ShapeDtypeStruct((n * DIM,), jnp.float32),
        scratch_types=[
            pltpu.VMEM_SHARED((NPAIR, PDIM), jnp.float32),
            pltpu.VMEM((W,), jnp.int32),
            pltpu.VMEM((W,), jnp.int32),
            pltpu.VMEM((W,), jnp.int32),
            pltpu.VMEM((W,), jnp.int32),
            pltpu.VMEM((W,), jnp.int32),
            pltpu.VMEM((W,), jnp.int32),
            pltpu.VMEM((W, PDIM), jnp.float32),
            pltpu.VMEM((W, PDIM), jnp.float32),
            pltpu.VMEM((W * DIM,), jnp.float32),
            pltpu.SemaphoreType.DMA,
            pltpu.SemaphoreType.DMA,
            pltpu.SemaphoreType.DMA,
            pltpu.SemaphoreType.DMA,
        ],
        compiler_params=pltpu.CompilerParams(use_tc_tiling_on_sc=False, needs_layout_passes=False),
    )
    def k(ids_hbm, table_hbm, out_hbm, table_sh,
          idx0, idx1, pidx0, pidx1, off0, off1, wide0, wide1, nar,
          isem0, isem1, gsem0, gsem1):
        cid = lax.axis_index("c")
        sid = lax.axis_index("s")
        nc = lax.axis_size("c")
        wid = sid * nc + cid
        base = wid * per_w

        r0 = sid * stage_rows
        pltpu.sync_copy(
            table_hbm.at[pl.ds(r0, stage_rows)],
            table_sh.at[pl.ds(r0, stage_rows)],
        )

        @pl.when(sid == 15)
        def _():
            t0 = 16 * stage_rows
            pltpu.sync_copy(
                table_hbm.at[pl.ds(t0, tail_rows)],
                table_sh.at[pl.ds(t0, tail_rows)],
            )

        plsc.subcore_barrier()

        idxb = (idx0, idx1)
        pidxb = (pidx0, pidx1)
        offb = (off0, off1)
        wideb = (wide0, wide1)
        isems = (isem0, isem1)
        gsems = (gsem0, gsem1)

        def split_pass(buf):
            def sp(q, carry):
                for u in range(4):
                    o = L * 4 * q + L * u
                    v = idxb[buf][pl.ds(o, L)]
                    pidxb[buf][pl.ds(o, L)] = lax.shift_right_logical(v, 1)
                    offb[buf][pl.ds(o, L)] = lax.shift_left(
                        lax.bitwise_and(v, 1), 2)
                return carry
            lax.fori_loop(0, W // (L * 4), sp, 0)

        rowp = lax.shift_right_logical(lax.iota(jnp.int32, L), 2)
        colp = lax.bitwise_and(lax.iota(jnp.int32, L), 3)

        def compact_pass(buf, wnd):
            def cp(q, carry):
                for u in range(4):
                    v = 4 * q + u
                    rows = rowp + 4 * v
                    cb = plsc.load_gather(offb[buf], [rows])
                    cols = cb + colp
                    g = plsc.load_gather(wideb[buf], [rows, cols])
                    nar[pl.ds(L * v, L)] = g
                return carry
            lax.fori_loop(0, W * DIM // (L * 4), cp, 0)
            pltpu.sync_copy(
                nar, out_hbm.at[pl.ds((base + wnd * W) * DIM, W * DIM)])

        pltpu.sync_copy(ids_hbm.at[pl.ds(base, W)], idx0)
        split_pass(0)
        pltpu.async_copy(table_sh.at[pidx0], wide0, gsem0)
        pltpu.async_copy(ids_hbm.at[pl.ds(base + W, W)], idx1, isem1)

        def body(it, carry):
            for kk in (0, 1):
                wnd = 2 * it + kk
                buf = kk
                nbuf = 1 - kk

                @pl.when(wnd + 1 < n_wnd)
                def _():
                    pltpu.make_async_copy(
                        ids_hbm.at[pl.ds(base + (wnd + 1) * W, W)],
                        idxb[nbuf], isems[nbuf]).wait()
                    split_pass(nbuf)
                    pltpu.async_copy(
                        table_sh.at[pidxb[nbuf]], wideb[nbuf], gsems[nbuf])

                @pl.when(wnd + 2 < n_wnd)
                def _():
                    pltpu.async_copy(
                        ids_hbm.at[pl.ds(base + (wnd + 2) * W, W)],
                        idxb[buf], isems[buf])

                pltpu.make_async_copy(
                    table_sh.at[pidxb[buf]], wideb[buf], gsems[buf]).wait()
                compact_pass(buf, wnd)
            return carry

        lax.fori_loop(0, n_wnd // 2, body, 0)

    return k


def kernel(src_input_ids, src_attention_mask, emb_table):
    del src_attention_mask
    b, s = src_input_ids.shape
    n = b * s
    assert n % (NW * W) == 0
    ids = src_input_ids.reshape(n).astype(jnp.int32)
    table_pairs = emb_table.reshape(NPAIR, PDIM)
    out = _emb_kernel(n)(ids, table_pairs)
    return out.reshape(b, s, DIM)
